# Initial kernel scaffold; baseline (speedup 1.0000x reference)
import jax
import jax.numpy as jnp
from jax.experimental import pallas as pl


def kernel(x, edge_index, batch_index, edge_attr, W0, b0, W1, b1, W2, b2, W3, b3, Wout, bout):
    raise NotImplementedError("write your pallas kernel here")


# speedup vs baseline: 1.7262x; 1.7262x over previous
import functools
import jax
import jax.numpy as jnp
from jax.experimental import pallas as pl
from jax.experimental.pallas import tpu as pltpu


def _matmul_kernel(x_ref, w_ref, o_ref):
    o_ref[...] = jnp.dot(x_ref[...], w_ref[...],
                         preferred_element_type=jnp.float32)


def _pallas_matmul(x, w, block_rows=2000):
    n, k = x.shape
    _, h = w.shape
    grid = n // block_rows
    return pl.pallas_call(
        _matmul_kernel,
        grid=(grid,),
        in_specs=[
            pl.BlockSpec((block_rows, k), lambda i: (i, 0)),
            pl.BlockSpec((k, h), lambda i: (0, 0)),
        ],
        out_specs=pl.BlockSpec((block_rows, h), lambda i: (i, 0)),
        out_shape=jax.ShapeDtypeStruct((n, h), jnp.float32),
    )(x, w)


def kernel(x, edge_index, batch_index, edge_attr, W0, b0, W1, b1, W2, b2, W3, b3, Wout, bout):
    n = x.shape[0]
    src = edge_index[0]
    dst = edge_index[1]
    deg = jnp.zeros((n,), jnp.float32).at[dst].add(edge_attr) + 1.0
    dinv = jax.lax.rsqrt(deg)

    def layer(h, W, b):
        xw = _pallas_matmul(h, W)
        y = xw * dinv[:, None]
        s = jnp.zeros((n, W.shape[1]), jnp.float32).at[dst].add(
            y[src] * edge_attr[:, None])
        return jnp.tanh(dinv[:, None] * (s + y) + b)

    h = layer(x, W0, b0)
    h = layer(h, W1, b1)
    h = layer(h, W2, b2)
    h = layer(h, W3, b3)

    g = 64
    counts = jnp.zeros((g,), jnp.float32).at[batch_index].add(1.0)
    mean_pool = jnp.zeros((g, h.shape[1]), jnp.float32).at[batch_index].add(h)
    mean_pool = mean_pool / jnp.maximum(counts, 1.0)[:, None]
    max_pool = jnp.full((g, h.shape[1]), -jnp.inf, jnp.float32).at[batch_index].max(h)
    max_pool = jnp.where(counts[:, None] > 0, max_pool, 0.0)
    pooled = jnp.concatenate([max_pool, mean_pool], axis=1)
    return (pooled @ Wout + bout).flatten()


# speedup vs baseline: 4.4372x; 2.5706x over previous
import functools
import jax
import jax.numpy as jnp
from jax import lax
from jax.experimental import pallas as pl
from jax.experimental.pallas import tpu as pltpu
from jax.experimental.pallas import tpu_sc as plsc

_N = 10000
_NP = 10240
_EPAD = 327680
_EROWS = _EPAD // 128
_H = 64
_G = 64
_NWORK = 32
_RW = _EROWS // _NWORK
_KC = 8
_NCH = _RW // _KC
_NPW = _NP // 16


def _sc_mesh():
    return plsc.VectorSubcoreMesh(core_axis_name="c", subcore_axis_name="s")


@functools.partial(
    pl.kernel,
    out_type=jax.ShapeDtypeStruct((2, _NP, 1), jnp.float32),
    mesh=_sc_mesh(),
    compiler_params=pltpu.CompilerParams(use_tc_tiling_on_sc=False),
    scratch_types=[
        pltpu.VMEM((_KC, 128), jnp.int32),
        pltpu.VMEM((_KC * 128, 1), jnp.float32),
        pltpu.VMEM_SHARED((_NP, 1), jnp.float32),
    ],
)
def _sc_deg(dst_hbm, wcol_hbm, zeros_hbm, out_hbm, dst_v, w_v, acc_sh):
    c = lax.axis_index("c")
    s = lax.axis_index("s")
    pltpu.sync_copy(zeros_hbm, acc_sh.at[pl.ds(s * _NPW, _NPW)])
    plsc.subcore_barrier()
    row0 = (s * 2 + c) * _RW

    def chunk(i, carry):
        r0 = row0 + i * _KC
        pltpu.sync_copy(dst_hbm.at[pl.ds(r0, _KC)], dst_v)
        pltpu.sync_copy(wcol_hbm.at[pl.ds(r0 * 128, _KC * 128)], w_v)
        for j in range(_KC):
            pltpu.sync_copy(w_v.at[pl.ds(j * 128, 128)],
                            acc_sh.at[dst_v.at[j]], add=True)
        return carry

    lax.fori_loop(0, _NCH, chunk, 0)
    plsc.subcore_barrier()
    pltpu.sync_copy(acc_sh.at[pl.ds(s * _NPW, _NPW)],
                    out_hbm.at[c].at[pl.ds(s * _NPW, _NPW)])


@functools.partial(
    pl.kernel,
    out_type=jax.ShapeDtypeStruct((2, _NP, _H), jnp.float32),
    mesh=_sc_mesh(),
    compiler_params=pltpu.CompilerParams(use_tc_tiling_on_sc=False),
    scratch_types=[
        pltpu.VMEM((_KC, 128), jnp.int32),
        pltpu.VMEM((_KC, 128), jnp.int32),
        pltpu.VMEM((_KC * 128,), jnp.float32),
        pltpu.VMEM((_KC * 128, _H), jnp.float32),
        pltpu.VMEM_SHARED((_NP, _H), jnp.float32),
        pltpu.SemaphoreType.DMA,
    ],
)
def _sc_edge(y_hbm, src_hbm, dst_hbm, w_hbm, zeros_hbm, out_hbm,
             src_v, dst_v, w_v, rows_v, acc_sh, sem):
    c = lax.axis_index("c")
    s = lax.axis_index("s")
    pltpu.sync_copy(zeros_hbm, acc_sh.at[pl.ds(s * _NPW, _NPW)])
    plsc.subcore_barrier()
    row0 = (s * 2 + c) * _RW

    def chunk(i, carry):
        r0 = row0 + i * _KC
        pltpu.sync_copy(src_hbm.at[pl.ds(r0, _KC)], src_v)
        pltpu.sync_copy(dst_hbm.at[pl.ds(r0, _KC)], dst_v)
        pltpu.sync_copy(w_hbm.at[pl.ds(r0 * 128, _KC * 128)], w_v)
        for j in range(_KC):
            pltpu.async_copy(y_hbm.at[src_v.at[j]],
                             rows_v.at[pl.ds(j * 128, 128)], sem).wait()

        def grp(t, carry2):
            base = t * 16
            w16 = w_v[pl.ds(base, 16)]
            for l in range(16):
                scale = lax.gather(
                    w16, jnp.full((16, 1), l, jnp.int32),
                    lax.GatherDimensionNumbers(
                        offset_dims=(), collapsed_slice_dims=(0,),
                        start_index_map=(0,)),
                    (1,), mode=lax.GatherScatterMode.PROMISE_IN_BOUNDS)
                for g in range(_H // 16):
                    cur = rows_v[pl.ds(base + l, 1), pl.ds(g * 16, 16)]
                    rows_v[pl.ds(base + l, 1), pl.ds(g * 16, 16)] = (
                        cur.reshape(16) * scale).reshape(1, 16)
            return carry2

        lax.fori_loop(0, _KC * 8, grp, 0)
        for j in range(_KC):
            pltpu.sync_copy(rows_v.at[pl.ds(j * 128, 128)],
                            acc_sh.at[dst_v.at[j]], add=True)
        return carry

    lax.fori_loop(0, _NCH, chunk, 0)
    plsc.subcore_barrier()
    pltpu.sync_copy(acc_sh.at[pl.ds(s * _NPW, _NPW)],
                    out_hbm.at[c].at[pl.ds(s * _NPW, _NPW)])


_BR = 2000


def _first_kernel(deg_ref, x_ref, w_ref, batch_ref, y_ref, dinv_ref, offs_ref):
    i = pl.program_id(0)
    deg = deg_ref[0] + deg_ref[1] + 1.0
    dinv = lax.rsqrt(deg)
    dinv_ref[...] = dinv
    xw = jnp.dot(x_ref[...], w_ref[...], preferred_element_type=jnp.float32)
    y_ref[...] = xw * dinv

    @pl.when(i == 0)
    def _():
        b = batch_ref[...]
        gids = lax.broadcasted_iota(jnp.int32, (1, 128), 1)
        offs_ref[...] = jnp.sum(
            (b < gids).astype(jnp.int32), axis=0, keepdims=True)


def _tc_first(deg_parts, x, w0, batch2d):
    grid = _N // _BR
    return pl.pallas_call(
        _first_kernel,
        grid=(grid,),
        in_specs=[
            pl.BlockSpec((2, _BR, 1), lambda i: (0, i, 0)),
            pl.BlockSpec((_BR, x.shape[1]), lambda i: (i, 0)),
            pl.BlockSpec(x.shape[1:] + (_H,), lambda i: (0, 0)),
            pl.BlockSpec((_N, 1), lambda i: (0, 0)),
        ],
        out_specs=[
            pl.BlockSpec((_BR, _H), lambda i: (i, 0)),
            pl.BlockSpec((_BR, 1), lambda i: (i, 0)),
            pl.BlockSpec((1, 128), lambda i: (0, 0)),
        ],
        out_shape=[
            jax.ShapeDtypeStruct((_N, _H), jnp.float32),
            jax.ShapeDtypeStruct((_N, 1), jnp.float32),
            jax.ShapeDtypeStruct((1, 128), jnp.int32),
        ],
    )(deg_parts, x, w0, batch2d)


def _layer_kernel(s_ref, y_ref, dinv_ref, b_ref, w_ref, o_ref):
    dinv = dinv_ref[...]
    h = jnp.tanh((s_ref[0] + s_ref[1] + y_ref[...]) * dinv + b_ref[...])
    o_ref[...] = jnp.dot(h, w_ref[...],
                         preferred_element_type=jnp.float32) * dinv


def _tc_layer(s_parts, y, dinv, b2d, w):
    grid = _N // _BR
    return pl.pallas_call(
        _layer_kernel,
        grid=(grid,),
        in_specs=[
            pl.BlockSpec((2, _BR, _H), lambda i: (0, i, 0)),
            pl.BlockSpec((_BR, _H), lambda i: (i, 0)),
            pl.BlockSpec((_BR, 1), lambda i: (i, 0)),
            pl.BlockSpec((1, _H), lambda i: (0, 0)),
            pl.BlockSpec((_H, _H), lambda i: (0, 0)),
        ],
        out_specs=pl.BlockSpec((_BR, _H), lambda i: (i, 0)),
        out_shape=jax.ShapeDtypeStruct((_N, _H), jnp.float32),
    )(s_parts, y, dinv, b2d, w)


def _pool_kernel(offs_ref, s_ref, y_ref, dinv_ref, b_ref, wout_ref, bout_ref,
                 o_ref, h_buf, pooled_buf):
    h = jnp.tanh((s_ref[0] + s_ref[1] + y_ref[...]) * dinv_ref[...]
                 + b_ref[...])
    h_buf[...] = h

    def group(g, carry):
        o0 = offs_ref[g]
        o1 = offs_ref[g + 1]
        cnt = o1 - o0
        nk = (cnt + 7) // 8

        def inner(k, mxsm):
            mx, sm = mxsm
            start = o0 + k * 8
            rows = h_buf[pl.ds(start, 8), :]
            rowid = start + lax.broadcasted_iota(jnp.int32, (8, _H), 0)
            valid = rowid < o1
            mx = jnp.maximum(mx, jnp.where(valid, rows, -jnp.inf))
            sm = sm + jnp.where(valid, rows, 0.0)
            return mx, sm

        mx, sm = lax.fori_loop(
            0, nk, inner,
            (jnp.full((8, _H), -jnp.inf, jnp.float32),
             jnp.zeros((8, _H), jnp.float32)))
        mxr = jnp.max(mx, axis=0, keepdims=True)
        smr = jnp.sum(sm, axis=0, keepdims=True)
        mean = smr / jnp.maximum(cnt.astype(jnp.float32), 1.0)
        mxr = jnp.where(cnt > 0, mxr, 0.0)
        pooled_buf[pl.ds(g, 1), 0:_H] = mxr
        pooled_buf[pl.ds(g, 1), _H:2 * _H] = mean
        return carry

    lax.fori_loop(0, _G, group, 0)
    o_ref[...] = jnp.dot(pooled_buf[...], wout_ref[...],
                         preferred_element_type=jnp.float32) + bout_ref[...]


def _tc_pool(offs, s_parts, y, dinv, b2d, wout, bout2d):
    return pl.pallas_call(
        _pool_kernel,
        in_specs=[
            pl.BlockSpec(memory_space=pltpu.SMEM),
            pl.BlockSpec((2, _N, _H), lambda: (0, 0, 0)),
            pl.BlockSpec((_N, _H), lambda: (0, 0)),
            pl.BlockSpec((_N, 1), lambda: (0, 0)),
            pl.BlockSpec((1, _H), lambda: (0, 0)),
            pl.BlockSpec((2 * _H, 1), lambda: (0, 0)),
            pl.BlockSpec((1, 1), lambda: (0, 0)),
        ],
        out_specs=pl.BlockSpec((_G, 1), lambda: (0, 0)),
        out_shape=jax.ShapeDtypeStruct((_G, 1), jnp.float32),
        scratch_shapes=[
            pltpu.VMEM((_N, _H), jnp.float32),
            pltpu.VMEM((_G, 2 * _H), jnp.float32),
        ],
    )(offs, s_parts, y, dinv, b2d, wout, bout2d)


def kernel(x, edge_index, batch_index, edge_attr, W0, b0, W1, b1, W2, b2, W3,
           b3, Wout, bout):
    npad = _EPAD - edge_attr.shape[0]
    src2d = jnp.concatenate(
        [edge_index[0], jnp.zeros((npad,), edge_index.dtype)]).reshape(
            _EROWS, 128).astype(jnp.int32)
    dst2d = jnp.concatenate(
        [edge_index[1], jnp.zeros((npad,), edge_index.dtype)]).reshape(
            _EROWS, 128).astype(jnp.int32)
    w_flat = jnp.concatenate(
        [edge_attr, jnp.zeros((npad,), jnp.float32)])
    w_col = w_flat.reshape(_EPAD, 1)
    zeros1 = jnp.zeros((_NPW, 1), jnp.float32)
    zeros64 = jnp.zeros((_NPW, _H), jnp.float32)
    batch2d = batch_index.astype(jnp.int32).reshape(_N, 1)

    deg_parts = _sc_deg(dst2d, w_col, zeros1)[:, :_N]
    y0, dinv, offs2d = _tc_first(deg_parts, x, W0, batch2d)
    offs = offs2d.reshape(128)

    s1 = _sc_edge(y0, src2d, dst2d, w_flat, zeros64)[:, :_N]
    y1 = _tc_layer(s1, y0, dinv, b0.reshape(1, _H), W1)
    s2 = _sc_edge(y1, src2d, dst2d, w_flat, zeros64)[:, :_N]
    y2 = _tc_layer(s2, y1, dinv, b1.reshape(1, _H), W2)
    s3 = _sc_edge(y2, src2d, dst2d, w_flat, zeros64)[:, :_N]
    y3 = _tc_layer(s3, y2, dinv, b2.reshape(1, _H), W3)
    s4 = _sc_edge(y3, src2d, dst2d, w_flat, zeros64)[:, :_N]

    out = _tc_pool(offs, s4, y3, dinv, b3.reshape(1, _H), Wout,
                   bout.reshape(1, 1))
    return out.reshape(_G)


# speedup vs baseline: 4.8030x; 1.0824x over previous
import functools
import jax
import jax.numpy as jnp
from jax import lax
from jax.experimental import pallas as pl
from jax.experimental.pallas import tpu as pltpu
from jax.experimental.pallas import tpu_sc as plsc

_N = 10000
_NP = 10240
_EPAD = 327680
_EROWS = _EPAD // 128
_H = 64
_G = 64
_NWORK = 32
_RW = _EROWS // _NWORK
_KC = 8
_NCH = _RW // _KC
_NPW = _NP // 16


def _sc_mesh():
    return plsc.VectorSubcoreMesh(core_axis_name="c", subcore_axis_name="s")


@functools.partial(
    pl.kernel,
    out_type=jax.ShapeDtypeStruct((2, _NP, 1), jnp.float32),
    mesh=_sc_mesh(),
    compiler_params=pltpu.CompilerParams(use_tc_tiling_on_sc=False),
    scratch_types=[
        pltpu.VMEM((_KC, 128), jnp.int32),
        pltpu.VMEM((_KC * 128, 1), jnp.float32),
        pltpu.VMEM_SHARED((_NP, 1), jnp.float32),
    ],
)
def _sc_deg(dst_hbm, wcol_hbm, zeros_hbm, out_hbm, dst_v, w_v, acc_sh):
    c = lax.axis_index("c")
    s = lax.axis_index("s")
    pltpu.sync_copy(zeros_hbm, acc_sh.at[pl.ds(s * _NPW, _NPW)])
    plsc.subcore_barrier()
    row0 = (s * 2 + c) * _RW

    def chunk(i, carry):
        r0 = row0 + i * _KC
        pltpu.sync_copy(dst_hbm.at[pl.ds(r0, _KC)], dst_v)
        pltpu.sync_copy(wcol_hbm.at[pl.ds(r0 * 128, _KC * 128)], w_v)
        for j in range(_KC):
            pltpu.sync_copy(w_v.at[pl.ds(j * 128, 128)],
                            acc_sh.at[dst_v.at[j]], add=True)
        return carry

    lax.fori_loop(0, _NCH, chunk, 0)
    plsc.subcore_barrier()
    pltpu.sync_copy(acc_sh.at[pl.ds(s * _NPW, _NPW)],
                    out_hbm.at[c].at[pl.ds(s * _NPW, _NPW)])


@functools.partial(
    pl.kernel,
    out_type=jax.ShapeDtypeStruct((2, _NP, _H), jnp.float32),
    mesh=_sc_mesh(),
    compiler_params=pltpu.CompilerParams(use_tc_tiling_on_sc=False),
    scratch_types=[
        pltpu.VMEM((_KC, 128), jnp.int32),
        pltpu.VMEM((_KC, 128), jnp.int32),
        pltpu.VMEM((_KC * 128,), jnp.float32),
        pltpu.VMEM((_KC * 64, _H), jnp.float32),
        pltpu.VMEM((_KC * 64, _H), jnp.float32),
        pltpu.SemaphoreType.DMA,
        pltpu.SemaphoreType.DMA,
        pltpu.SemaphoreType.DMA,
        pltpu.SemaphoreType.DMA,
        pltpu.VMEM_SHARED((_NP, _H), jnp.float32),
    ],
)
def _sc_edge(y_hbm, src_hbm, dst_hbm, w_hbm, zeros_hbm, out_hbm,
             src_v, dst_v, w_v, rows_a, rows_b, gsem_a, gsem_b,
             ssem_a, ssem_b, acc_sh):
    c = lax.axis_index("c")
    s = lax.axis_index("s")
    pltpu.sync_copy(zeros_hbm, acc_sh.at[pl.ds(s * _NPW, _NPW)])
    plsc.subcore_barrier()
    row0 = (s * 2 + c) * _RW
    half = _KC // 2

    def scale_half(rows_v, wbase):
        def grp(t, carry2):
            base = t * 16
            w16 = w_v[pl.ds(wbase + base, 16)]
            for l in range(16):
                scale = lax.gather(
                    w16, jnp.full((16, 1), l, jnp.int32),
                    lax.GatherDimensionNumbers(
                        offset_dims=(), collapsed_slice_dims=(0,),
                        start_index_map=(0,)),
                    (1,), mode=lax.GatherScatterMode.PROMISE_IN_BOUNDS)
                for g in range(_H // 16):
                    cur = rows_v[pl.ds(base + l, 1), pl.ds(g * 16, 16)]
                    rows_v[pl.ds(base + l, 1), pl.ds(g * 16, 16)] = (
                        cur.reshape(16) * scale).reshape(1, 16)
            return carry2

        lax.fori_loop(0, _KC * 4, grp, 0)

    def drain_scatters(rows_v, ssem):
        for j in range(half):
            pltpu.make_async_copy(
                y_hbm.at[pl.ds(0, 128)],
                rows_v.at[pl.ds(j * 128, 128)], ssem).wait()

    def chunk(i, carry):
        r0 = row0 + i * _KC

        @pl.when(i > 0)
        def _():
            drain_scatters(rows_a, ssem_a)
            drain_scatters(rows_b, ssem_b)

        pltpu.sync_copy(src_hbm.at[pl.ds(r0, _KC)], src_v)
        pltpu.sync_copy(dst_hbm.at[pl.ds(r0, _KC)], dst_v)
        pltpu.sync_copy(w_hbm.at[pl.ds(r0 * 128, _KC * 128)], w_v)

        ga = [pltpu.async_copy(y_hbm.at[src_v.at[j]],
                               rows_a.at[pl.ds(j * 128, 128)], gsem_a)
              for j in range(half)]
        gb = [pltpu.async_copy(y_hbm.at[src_v.at[half + j]],
                               rows_b.at[pl.ds(j * 128, 128)], gsem_b)
              for j in range(half)]
        for d in ga:
            d.wait()
        scale_half(rows_a, 0)
        for j in range(half):
            pltpu.async_copy(rows_a.at[pl.ds(j * 128, 128)],
                             acc_sh.at[dst_v.at[j]], ssem_a, add=True)
        for d in gb:
            d.wait()
        scale_half(rows_b, half * 128)
        for j in range(half):
            pltpu.async_copy(rows_b.at[pl.ds(j * 128, 128)],
                             acc_sh.at[dst_v.at[half + j]], ssem_b, add=True)
        return carry

    lax.fori_loop(0, _NCH, chunk, 0)
    drain_scatters(rows_a, ssem_a)
    drain_scatters(rows_b, ssem_b)
    plsc.subcore_barrier()
    pltpu.sync_copy(acc_sh.at[pl.ds(s * _NPW, _NPW)],
                    out_hbm.at[c].at[pl.ds(s * _NPW, _NPW)])


_BR = 2000


def _first_kernel(deg_ref, x_ref, w_ref, batch_ref, y_ref, dinv_ref, offs_ref):
    i = pl.program_id(0)
    deg = deg_ref[0] + deg_ref[1] + 1.0
    dinv = lax.rsqrt(deg)
    dinv_ref[...] = dinv
    xw = jnp.dot(x_ref[...], w_ref[...], preferred_element_type=jnp.float32)
    y_ref[...] = xw * dinv

    @pl.when(i == 0)
    def _():
        b = batch_ref[...]
        gids = lax.broadcasted_iota(jnp.int32, (1, 128), 1)
        offs_ref[...] = jnp.sum(
            (b < gids).astype(jnp.int32), axis=0, keepdims=True)


def _tc_first(deg_parts, x, w0, batch2d):
    grid = _N // _BR
    return pl.pallas_call(
        _first_kernel,
        grid=(grid,),
        in_specs=[
            pl.BlockSpec((2, _BR, 1), lambda i: (0, i, 0)),
            pl.BlockSpec((_BR, x.shape[1]), lambda i: (i, 0)),
            pl.BlockSpec(x.shape[1:] + (_H,), lambda i: (0, 0)),
            pl.BlockSpec((_N, 1), lambda i: (0, 0)),
        ],
        out_specs=[
            pl.BlockSpec((_BR, _H), lambda i: (i, 0)),
            pl.BlockSpec((_BR, 1), lambda i: (i, 0)),
            pl.BlockSpec((1, 128), lambda i: (0, 0)),
        ],
        out_shape=[
            jax.ShapeDtypeStruct((_N, _H), jnp.float32),
            jax.ShapeDtypeStruct((_N, 1), jnp.float32),
            jax.ShapeDtypeStruct((1, 128), jnp.int32),
        ],
    )(deg_parts, x, w0, batch2d)


def _layer_kernel(s_ref, y_ref, dinv_ref, b_ref, w_ref, o_ref):
    dinv = dinv_ref[...]
    h = jnp.tanh((s_ref[0] + s_ref[1] + y_ref[...]) * dinv + b_ref[...])
    o_ref[...] = jnp.dot(h, w_ref[...],
                         preferred_element_type=jnp.float32) * dinv


def _tc_layer(s_parts, y, dinv, b2d, w):
    grid = _N // _BR
    return pl.pallas_call(
        _layer_kernel,
        grid=(grid,),
        in_specs=[
            pl.BlockSpec((2, _BR, _H), lambda i: (0, i, 0)),
            pl.BlockSpec((_BR, _H), lambda i: (i, 0)),
            pl.BlockSpec((_BR, 1), lambda i: (i, 0)),
            pl.BlockSpec((1, _H), lambda i: (0, 0)),
            pl.BlockSpec((_H, _H), lambda i: (0, 0)),
        ],
        out_specs=pl.BlockSpec((_BR, _H), lambda i: (i, 0)),
        out_shape=jax.ShapeDtypeStruct((_N, _H), jnp.float32),
    )(s_parts, y, dinv, b2d, w)


def _pool_kernel(offs_ref, s_ref, y_ref, dinv_ref, b_ref, wout_ref, bout_ref,
                 o_ref, h_buf, pooled_buf):
    h = jnp.tanh((s_ref[0] + s_ref[1] + y_ref[...]) * dinv_ref[...]
                 + b_ref[...])
    h_buf[...] = h

    def group(g, carry):
        o0 = offs_ref[g]
        o1 = offs_ref[g + 1]
        cnt = o1 - o0
        nk = (cnt + 7) // 8

        def inner(k, mxsm):
            mx, sm = mxsm
            start = o0 + k * 8
            rows = h_buf[pl.ds(start, 8), :]
            rowid = start + lax.broadcasted_iota(jnp.int32, (8, _H), 0)
            valid = rowid < o1
            mx = jnp.maximum(mx, jnp.where(valid, rows, -jnp.inf))
            sm = sm + jnp.where(valid, rows, 0.0)
            return mx, sm

        mx, sm = lax.fori_loop(
            0, nk, inner,
            (jnp.full((8, _H), -jnp.inf, jnp.float32),
             jnp.zeros((8, _H), jnp.float32)))
        mxr = jnp.max(mx, axis=0, keepdims=True)
        smr = jnp.sum(sm, axis=0, keepdims=True)
        mean = smr / jnp.maximum(cnt.astype(jnp.float32), 1.0)
        mxr = jnp.where(cnt > 0, mxr, 0.0)
        pooled_buf[pl.ds(g, 1), 0:_H] = mxr
        pooled_buf[pl.ds(g, 1), _H:2 * _H] = mean
        return carry

    lax.fori_loop(0, _G, group, 0)
    o_ref[...] = jnp.dot(pooled_buf[...], wout_ref[...],
                         preferred_element_type=jnp.float32) + bout_ref[...]


def _tc_pool(offs, s_parts, y, dinv, b2d, wout, bout2d):
    return pl.pallas_call(
        _pool_kernel,
        in_specs=[
            pl.BlockSpec(memory_space=pltpu.SMEM),
            pl.BlockSpec((2, _N, _H), lambda: (0, 0, 0)),
            pl.BlockSpec((_N, _H), lambda: (0, 0)),
            pl.BlockSpec((_N, 1), lambda: (0, 0)),
            pl.BlockSpec((1, _H), lambda: (0, 0)),
            pl.BlockSpec((2 * _H, 1), lambda: (0, 0)),
            pl.BlockSpec((1, 1), lambda: (0, 0)),
        ],
        out_specs=pl.BlockSpec((_G, 1), lambda: (0, 0)),
        out_shape=jax.ShapeDtypeStruct((_G, 1), jnp.float32),
        scratch_shapes=[
            pltpu.VMEM((_N, _H), jnp.float32),
            pltpu.VMEM((_G, 2 * _H), jnp.float32),
        ],
    )(offs, s_parts, y, dinv, b2d, wout, bout2d)


def kernel(x, edge_index, batch_index, edge_attr, W0, b0, W1, b1, W2, b2, W3,
           b3, Wout, bout):
    npad = _EPAD - edge_attr.shape[0]
    src2d = jnp.concatenate(
        [edge_index[0], jnp.zeros((npad,), edge_index.dtype)]).reshape(
            _EROWS, 128).astype(jnp.int32)
    dst2d = jnp.concatenate(
        [edge_index[1], jnp.zeros((npad,), edge_index.dtype)]).reshape(
            _EROWS, 128).astype(jnp.int32)
    w_flat = jnp.concatenate(
        [edge_attr, jnp.zeros((npad,), jnp.float32)])
    w_col = w_flat.reshape(_EPAD, 1)
    zeros1 = jnp.zeros((_NPW, 1), jnp.float32)
    zeros64 = jnp.zeros((_NPW, _H), jnp.float32)
    batch2d = batch_index.astype(jnp.int32).reshape(_N, 1)

    deg_parts = _sc_deg(dst2d, w_col, zeros1)[:, :_N]
    y0, dinv, offs2d = _tc_first(deg_parts, x, W0, batch2d)
    offs = offs2d.reshape(128)

    s1 = _sc_edge(y0, src2d, dst2d, w_flat, zeros64)[:, :_N]
    y1 = _tc_layer(s1, y0, dinv, b0.reshape(1, _H), W1)
    s2 = _sc_edge(y1, src2d, dst2d, w_flat, zeros64)[:, :_N]
    y2 = _tc_layer(s2, y1, dinv, b1.reshape(1, _H), W2)
    s3 = _sc_edge(y2, src2d, dst2d, w_flat, zeros64)[:, :_N]
    y3 = _tc_layer(s3, y2, dinv, b2.reshape(1, _H), W3)
    s4 = _sc_edge(y3, src2d, dst2d, w_flat, zeros64)[:, :_N]

    out = _tc_pool(offs, s4, y3, dinv, b3.reshape(1, _H), Wout,
                   bout.reshape(1, 1))
    return out.reshape(_G)


# speedup vs baseline: 7.7260x; 1.6086x over previous
import functools
import jax
import jax.numpy as jnp
from jax import lax
from jax.experimental import pallas as pl
from jax.experimental.pallas import tpu as pltpu
from jax.experimental.pallas import tpu_sc as plsc

_N = 10000
_NP = 10240
_EPAD = 327680
_EROWS = _EPAD // 128
_H = 64
_G = 64
_RW = _EROWS // 32
_KC = 8
_NCH = _RW // _KC
_QR = 2
_NPW = _NP // 16


def _sc_mesh():
    return plsc.VectorSubcoreMesh(core_axis_name="c", subcore_axis_name="s")


@functools.partial(
    pl.kernel,
    out_type=jax.ShapeDtypeStruct((2, _NP, 1), jnp.float32),
    mesh=_sc_mesh(),
    compiler_params=pltpu.CompilerParams(use_tc_tiling_on_sc=False),
    scratch_types=[
        pltpu.VMEM((_KC, 128), jnp.int32),
        pltpu.VMEM((_KC * 128, 1), jnp.float32),
        pltpu.VMEM_SHARED((_NP, 1), jnp.float32),
    ],
)
def _sc_deg(dst_hbm, wcol_hbm, zeros_hbm, out_hbm, dst_v, w_v, acc_sh):
    c = lax.axis_index("c")
    s = lax.axis_index("s")
    pltpu.sync_copy(zeros_hbm, acc_sh.at[pl.ds(s * _NPW, _NPW)])
    plsc.subcore_barrier()
    row0 = (s * 2 + c) * _RW

    def chunk(i, carry):
        r0 = row0 + i * _KC
        pltpu.sync_copy(dst_hbm.at[pl.ds(r0, _KC)], dst_v)
        pltpu.sync_copy(wcol_hbm.at[pl.ds(r0 * 128, _KC * 128)], w_v)
        for j in range(_KC):
            pltpu.sync_copy(w_v.at[pl.ds(j * 128, 128)],
                            acc_sh.at[dst_v.at[j]], add=True)
        return carry

    lax.fori_loop(0, _NCH, chunk, 0)
    plsc.subcore_barrier()
    pltpu.sync_copy(acc_sh.at[pl.ds(s * _NPW, _NPW)],
                    out_hbm.at[c].at[pl.ds(s * _NPW, _NPW)])


@functools.partial(
    pl.kernel,
    out_type=jax.ShapeDtypeStruct((2, _NP, _H), jnp.float32),
    mesh=_sc_mesh(),
    compiler_params=pltpu.CompilerParams(use_tc_tiling_on_sc=False),
    scratch_types=[
        pltpu.VMEM((_KC, 128), jnp.int32),
        pltpu.VMEM((_KC, 128), jnp.int32),
        pltpu.VMEM((_KC * 128,), jnp.float32),
        pltpu.VMEM((_QR * 128, _H), jnp.float32),
        pltpu.VMEM((_QR * 128, _H), jnp.float32),
        pltpu.SemaphoreType.DMA,
        pltpu.SemaphoreType.DMA,
        pltpu.SemaphoreType.DMA,
        pltpu.SemaphoreType.DMA,
        pltpu.VMEM_SHARED((_NP, _H), jnp.float32),
        pltpu.VMEM_SHARED((_NP, _H), jnp.float32),
    ],
)
def _sc_edge(y_hbm, src_hbm, dst_hbm, w_hbm, zeros_hbm, out_hbm,
             src_v, dst_v, w_v, rows_a, rows_b, gsem_a, gsem_b,
             ssem_a, ssem_b, acc_sh, y_sh):
    c = lax.axis_index("c")
    s = lax.axis_index("s")
    pltpu.sync_copy(zeros_hbm, acc_sh.at[pl.ds(s * _NPW, _NPW)])
    pltpu.sync_copy(y_hbm.at[pl.ds(s * _NPW, _NPW)],
                    y_sh.at[pl.ds(s * _NPW, _NPW)])
    plsc.subcore_barrier()
    row0 = (s * 2 + c) * _RW

    def scale_rows(rows_v, wbase):
        def grp(t, carry2):
            base = t * 16
            w16 = w_v[pl.ds(wbase + base, 16)]
            for l in range(16):
                scale = lax.gather(
                    w16, jnp.full((16, 1), l, jnp.int32),
                    lax.GatherDimensionNumbers(
                        offset_dims=(), collapsed_slice_dims=(0,),
                        start_index_map=(0,)),
                    (1,), mode=lax.GatherScatterMode.PROMISE_IN_BOUNDS)
                for g in range(_H // 16):
                    cur = rows_v[pl.ds(base + l, 1), pl.ds(g * 16, 16)]
                    rows_v[pl.ds(base + l, 1), pl.ds(g * 16, 16)] = (
                        cur.reshape(16) * scale).reshape(1, 16)
            return carry2

        lax.fori_loop(0, _QR * 8, grp, 0)

    def fire_gathers(rows_v, gsem, jrows):
        return [pltpu.async_copy(y_sh.at[src_v.at[j]],
                                 rows_v.at[pl.ds(k * 128, 128)], gsem)
                for k, j in enumerate(jrows)]

    def fire_scatters(rows_v, ssem, jrows):
        for k, j in enumerate(jrows):
            pltpu.async_copy(rows_v.at[pl.ds(k * 128, 128)],
                             acc_sh.at[dst_v.at[j]], ssem, add=True)

    def drain_scatters(rows_v, ssem):
        for k in range(_QR):
            pltpu.make_async_copy(
                y_hbm.at[pl.ds(0, 128)],
                rows_v.at[pl.ds(k * 128, 128)], ssem).wait()

    def quarter(rows_v, gsem, ssem, jrows, descs):
        for d in descs:
            d.wait()
        scale_rows(rows_v, jrows[0] * 128)
        fire_scatters(rows_v, ssem, jrows)

    def chunk(i, carry):
        r0 = row0 + i * _KC

        @pl.when(i > 0)
        def _():
            drain_scatters(rows_a, ssem_a)
            drain_scatters(rows_b, ssem_b)

        pltpu.sync_copy(src_hbm.at[pl.ds(r0, _KC)], src_v)
        pltpu.sync_copy(dst_hbm.at[pl.ds(r0, _KC)], dst_v)
        pltpu.sync_copy(w_hbm.at[pl.ds(r0 * 128, _KC * 128)], w_v)

        g0 = fire_gathers(rows_a, gsem_a, (0, 1))
        g1 = fire_gathers(rows_b, gsem_b, (2, 3))
        quarter(rows_a, gsem_a, ssem_a, (0, 1), g0)
        quarter(rows_b, gsem_b, ssem_b, (2, 3), g1)
        drain_scatters(rows_a, ssem_a)
        g2 = fire_gathers(rows_a, gsem_a, (4, 5))
        drain_scatters(rows_b, ssem_b)
        g3 = fire_gathers(rows_b, gsem_b, (6, 7))
        quarter(rows_a, gsem_a, ssem_a, (4, 5), g2)
        quarter(rows_b, gsem_b, ssem_b, (6, 7), g3)
        return carry

    lax.fori_loop(0, _NCH, chunk, 0)
    drain_scatters(rows_a, ssem_a)
    drain_scatters(rows_b, ssem_b)
    plsc.subcore_barrier()
    pltpu.sync_copy(acc_sh.at[pl.ds(s * _NPW, _NPW)],
                    out_hbm.at[c].at[pl.ds(s * _NPW, _NPW)])


_BR = 2048


def _first_kernel(deg_ref, x_ref, w_ref, batch_ref, y_ref, dinv_ref, offs_ref):
    i = pl.program_id(0)
    deg = deg_ref[0] + deg_ref[1] + 1.0
    dinv = lax.rsqrt(deg)
    dinv_ref[...] = dinv
    xw = jnp.dot(x_ref[...], w_ref[...], preferred_element_type=jnp.float32)
    y_ref[...] = xw * dinv

    @pl.when(i == 0)
    def _():
        b = batch_ref[...]
        gids = lax.broadcasted_iota(jnp.int32, (1, 128), 1)
        offs_ref[...] = jnp.sum(
            (b < gids).astype(jnp.int32), axis=0, keepdims=True)


def _tc_first(deg_parts, x, w0, batch2d):
    grid = _NP // _BR
    return pl.pallas_call(
        _first_kernel,
        grid=(grid,),
        in_specs=[
            pl.BlockSpec((2, _BR, 1), lambda i: (0, i, 0)),
            pl.BlockSpec((_BR, x.shape[1]), lambda i: (i, 0)),
            pl.BlockSpec(x.shape[1:] + (_H,), lambda i: (0, 0)),
            pl.BlockSpec((_NP, 1), lambda i: (0, 0)),
        ],
        out_specs=[
            pl.BlockSpec((_BR, _H), lambda i: (i, 0)),
            pl.BlockSpec((_BR, 1), lambda i: (i, 0)),
            pl.BlockSpec((1, 128), lambda i: (0, 0)),
        ],
        out_shape=[
            jax.ShapeDtypeStruct((_NP, _H), jnp.float32),
            jax.ShapeDtypeStruct((_NP, 1), jnp.float32),
            jax.ShapeDtypeStruct((1, 128), jnp.int32),
        ],
    )(deg_parts, x, w0, batch2d)


def _layer_kernel(s_ref, y_ref, dinv_ref, b_ref, w_ref, o_ref):
    dinv = dinv_ref[...]
    h = jnp.tanh((s_ref[0] + s_ref[1] + y_ref[...]) * dinv + b_ref[...])
    o_ref[...] = jnp.dot(h, w_ref[...],
                         preferred_element_type=jnp.float32) * dinv


def _tc_layer(s_parts, y, dinv, b2d, w):
    grid = _NP // _BR
    return pl.pallas_call(
        _layer_kernel,
        grid=(grid,),
        in_specs=[
            pl.BlockSpec((2, _BR, _H), lambda i: (0, i, 0)),
            pl.BlockSpec((_BR, _H), lambda i: (i, 0)),
            pl.BlockSpec((_BR, 1), lambda i: (i, 0)),
            pl.BlockSpec((1, _H), lambda i: (0, 0)),
            pl.BlockSpec((_H, _H), lambda i: (0, 0)),
        ],
        out_specs=pl.BlockSpec((_BR, _H), lambda i: (i, 0)),
        out_shape=jax.ShapeDtypeStruct((_NP, _H), jnp.float32),
    )(s_parts, y, dinv, b2d, w)


def _pool_kernel(offs_ref, s_ref, y_ref, dinv_ref, b_ref, wout_ref, bout_ref,
                 o_ref, h_buf, pooled_buf):
    h = jnp.tanh((s_ref[0] + s_ref[1] + y_ref[...]) * dinv_ref[...]
                 + b_ref[...])
    h_buf[...] = h

    def group(g, carry):
        o0 = offs_ref[g]
        o1 = offs_ref[g + 1]
        cnt = o1 - o0
        nk = (cnt + 7) // 8

        def inner(k, mxsm):
            mx, sm = mxsm
            start = o0 + k * 8
            rows = h_buf[pl.ds(start, 8), :]
            rowid = start + lax.broadcasted_iota(jnp.int32, (8, _H), 0)
            valid = rowid < o1
            mx = jnp.maximum(mx, jnp.where(valid, rows, -jnp.inf))
            sm = sm + jnp.where(valid, rows, 0.0)
            return mx, sm

        mx, sm = lax.fori_loop(
            0, nk, inner,
            (jnp.full((8, _H), -jnp.inf, jnp.float32),
             jnp.zeros((8, _H), jnp.float32)))
        mxr = jnp.max(mx, axis=0, keepdims=True)
        smr = jnp.sum(sm, axis=0, keepdims=True)
        mean = smr / jnp.maximum(cnt.astype(jnp.float32), 1.0)
        mxr = jnp.where(cnt > 0, mxr, 0.0)
        pooled_buf[pl.ds(g, 1), 0:_H] = mxr
        pooled_buf[pl.ds(g, 1), _H:2 * _H] = mean
        return carry

    lax.fori_loop(0, _G, group, 0)
    o_ref[...] = jnp.dot(pooled_buf[...], wout_ref[...],
                         preferred_element_type=jnp.float32) + bout_ref[...]


def _tc_pool(offs, s_parts, y, dinv, b2d, wout, bout2d):
    return pl.pallas_call(
        _pool_kernel,
        in_specs=[
            pl.BlockSpec(memory_space=pltpu.SMEM),
            pl.BlockSpec((2, _NP, _H), lambda: (0, 0, 0)),
            pl.BlockSpec((_NP, _H), lambda: (0, 0)),
            pl.BlockSpec((_NP, 1), lambda: (0, 0)),
            pl.BlockSpec((1, _H), lambda: (0, 0)),
            pl.BlockSpec((2 * _H, 1), lambda: (0, 0)),
            pl.BlockSpec((1, 1), lambda: (0, 0)),
        ],
        out_specs=pl.BlockSpec((_G, 1), lambda: (0, 0)),
        out_shape=jax.ShapeDtypeStruct((_G, 1), jnp.float32),
        scratch_shapes=[
            pltpu.VMEM((_NP, _H), jnp.float32),
            pltpu.VMEM((_G, 2 * _H), jnp.float32),
        ],
    )(offs, s_parts, y, dinv, b2d, wout, bout2d)


def kernel(x, edge_index, batch_index, edge_attr, W0, b0, W1, b1, W2, b2, W3,
           b3, Wout, bout):
    npad = _EPAD - edge_attr.shape[0]
    src2d = jnp.concatenate(
        [edge_index[0], jnp.zeros((npad,), edge_index.dtype)]).reshape(
            _EROWS, 128).astype(jnp.int32)
    dst2d = jnp.concatenate(
        [edge_index[1], jnp.zeros((npad,), edge_index.dtype)]).reshape(
            _EROWS, 128).astype(jnp.int32)
    w_flat = jnp.concatenate([edge_attr, jnp.zeros((npad,), jnp.float32)])
    w_col = w_flat.reshape(_EPAD, 1)
    zeros1 = jnp.zeros((_NPW, 1), jnp.float32)
    zeros64 = jnp.zeros((_NPW, _H), jnp.float32)
    xp = jnp.concatenate(
        [x, jnp.zeros((_NP - _N, x.shape[1]), jnp.float32)])
    batch2d = jnp.concatenate(
        [batch_index.astype(jnp.int32),
         jnp.full((_NP - _N,), 127, jnp.int32)]).reshape(_NP, 1)

    deg_parts = _sc_deg(dst2d, w_col, zeros1)
    y0, dinv, offs2d = _tc_first(deg_parts, xp, W0, batch2d)
    offs = offs2d.reshape(128)

    s1 = _sc_edge(y0, src2d, dst2d, w_flat, zeros64)
    y1 = _tc_layer(s1, y0, dinv, b0.reshape(1, _H), W1)
    s2 = _sc_edge(y1, src2d, dst2d, w_flat, zeros64)
    y2 = _tc_layer(s2, y1, dinv, b1.reshape(1, _H), W2)
    s3 = _sc_edge(y2, src2d, dst2d, w_flat, zeros64)
    y3 = _tc_layer(s3, y2, dinv, b2.reshape(1, _H), W3)
    s4 = _sc_edge(y3, src2d, dst2d, w_flat, zeros64)

    out = _tc_pool(offs, s4, y3, dinv, b3.reshape(1, _H), Wout,
                   bout.reshape(1, 1))
    return out.reshape(_G)
